# baseline (speedup 1.0000x reference)
import functools

import jax
import jax.numpy as jnp
from jax import lax
from jax.experimental import pallas as pl
from jax.experimental.pallas import tpu as pltpu
from jax.experimental.pallas import tpu_sc as plsc

CHUNK = 32
NBUF = 3


@functools.lru_cache(maxsize=None)
def _make_sc_kernel(batch, seq_length, embed_dim, dtype):
    info = plsc.get_sparse_core_info()
    num_workers = info.num_cores * info.num_subcores
    rows_per_w = seq_length // num_workers
    n_chunks = rows_per_w // CHUNK

    mesh = plsc.VectorSubcoreMesh(core_axis_name="c", subcore_axis_name="s")

    scratch = [pltpu.VMEM((CHUNK, embed_dim), dtype) for _ in range(NBUF)]
    scratch += [pltpu.SemaphoreType.DMA for _ in range(2 * NBUF)]

    @functools.partial(
        pl.kernel,
        mesh=mesh,
        out_type=jax.ShapeDtypeStruct((batch, seq_length, embed_dim), dtype),
        scratch_types=scratch,
    )
    def k(emb_hbm, out_hbm, *bufs_and_sems):
        bufs = bufs_and_sems[:NBUF]
        rsems = bufs_and_sems[NBUF:2 * NBUF]
        wsems = bufs_and_sems[2 * NBUF:]
        wid = lax.axis_index("s") * info.num_cores + lax.axis_index("c")
        base = wid * rows_per_w

        def read(ci):
            b = ci % NBUF
            return pltpu.async_copy(
                emb_hbm.at[pl.ds(base + ci * CHUNK, CHUNK)], bufs[b],
                rsems[b])

        reads = {0: read(0)}
        writes = {}
        for ci in range(n_chunks):
            b = ci % NBUF
            reads.pop(ci).wait()
            if ci + 1 < n_chunks:
                nb = (ci + 1) % NBUF
                if ci + 1 >= NBUF:
                    for h in writes.pop(ci + 1 - NBUF):
                        h.wait()
                reads[ci + 1] = read(ci + 1)
            writes[ci] = [
                pltpu.async_copy(
                    bufs[b],
                    out_hbm.at[bb, pl.ds(base + ci * CHUNK, CHUNK)],
                    wsems[b])
                for bb in range(batch)
            ]
        for ci in sorted(writes):
            for h in writes[ci]:
                h.wait()

    return k


def kernel(input_ids, embedding):
    batch, seq_length = input_ids.shape
    k = _make_sc_kernel(batch, seq_length, embedding.shape[1],
                        embedding.dtype)
    return k(embedding)


# speedup vs baseline: 1.0180x; 1.0180x over previous
import functools

import jax
import jax.numpy as jnp
from jax import lax
from jax.experimental import pallas as pl
from jax.experimental.pallas import tpu as pltpu
from jax.experimental.pallas import tpu_sc as plsc

CHUNK = 32


@functools.lru_cache(maxsize=None)
def _make_sc_kernel(batch, seq_length, embed_dim, dtype):
    info = plsc.get_sparse_core_info()
    num_cores, num_sub = info.num_cores, info.num_subcores
    num_workers = num_cores * num_sub
    rows_per_w = seq_length // num_workers
    half = rows_per_w // 2
    n_chunks = half // CHUNK

    mesh = plsc.VectorSubcoreMesh(core_axis_name="c", subcore_axis_name="s")

    scratch = [pltpu.VMEM((n_chunks, CHUNK, embed_dim), dtype),
               pltpu.VMEM_SHARED((num_sub, n_chunks, CHUNK, embed_dim),
                                 dtype)]
    scratch += [pltpu.SemaphoreType.DMA for _ in range(4)]

    @functools.partial(
        pl.kernel,
        mesh=mesh,
        out_type=jax.ShapeDtypeStruct((batch, seq_length, embed_dim), dtype),
        scratch_types=scratch,
    )
    def k(emb_hbm, out_hbm, tbuf, sbuf, rsem_a, rsem_b, wsem_a, wsem_b):
        sid = lax.axis_index("s")
        wid = sid * num_cores + lax.axis_index("c")
        base = wid * rows_per_w

        ra, rb = [], []
        for ci in range(n_chunks):
            ra.append(pltpu.async_copy(
                emb_hbm.at[pl.ds(base + ci * CHUNK, CHUNK)],
                tbuf.at[ci], rsem_a))
            rb.append(pltpu.async_copy(
                emb_hbm.at[pl.ds(base + half + ci * CHUNK, CHUNK)],
                sbuf.at[sid, ci], rsem_b))
        writes = []
        for ci in range(n_chunks):
            ra[ci].wait()
            for bb in range(batch):
                writes.append(pltpu.async_copy(
                    tbuf.at[ci],
                    out_hbm.at[bb, pl.ds(base + ci * CHUNK, CHUNK)],
                    wsem_a))
            rb[ci].wait()
            for bb in range(batch):
                writes.append(pltpu.async_copy(
                    sbuf.at[sid, ci],
                    out_hbm.at[bb, pl.ds(base + half + ci * CHUNK, CHUNK)],
                    wsem_b))
        for h in writes:
            h.wait()

    return k


def kernel(input_ids, embedding):
    batch, seq_length = input_ids.shape
    k = _make_sc_kernel(batch, seq_length, embedding.shape[1],
                        embedding.dtype)
    return k(embedding)
